# baseline (speedup 1.0000x reference)
import functools

import jax
import jax.numpy as jnp
from jax import lax
from jax.experimental import pallas as pl
from jax.experimental.pallas import tpu as pltpu

TEMPERATURE = 8.0
NUM_SAMPLED = 1024
EPS = 1e-06

_SUB = 64
_LANE = 128


@functools.cache
def _gumbel(B, H, S):
    g = jax.random.gumbel(jax.random.key(42), (B, H, S), dtype=jnp.float32)
    return g.reshape(B * H, S // _LANE, _LANE)


def _row_body(k_ref, q_ref, g_ref, tm_ref, o_ref):
    S = _SUB * _LANE
    kb = k_ref[0].astype(jnp.bfloat16).astype(jnp.float32)
    qv = q_ref[0].astype(jnp.bfloat16).astype(jnp.float32)
    attn = jnp.sum(kb * qv[None], axis=-1) / TEMPERATURE
    m = jnp.max(attn)
    e = jnp.exp(attn - m) * tm_ref[0]
    se = jnp.sum(e)
    p = (e + EPS / S) / (se + EPS)
    sc = jnp.log(p) + g_ref[0]

    ki = lax.bitcast_convert_type(sc, jnp.int32)
    t = ki ^ ((ki >> 31) & jnp.int32(0x7FFFFFFF))
    u = lax.bitcast_convert_type(t, jnp.uint32) ^ jnp.uint32(0x80000000)

    T = jnp.uint32(0)
    for b in range(31, -1, -1):
        cand = T | jnp.uint32(1 << b)
        cnt = jnp.sum((u >= cand).astype(jnp.int32))
        T = jnp.where(cnt >= NUM_SAMPLED, cand, T)

    mask = (u >= T).astype(jnp.int32)

    a = pltpu.roll(mask, 1, 1)
    c = pltpu.roll(a, 1, 0)
    lane = lax.broadcasted_iota(jnp.int32, (_SUB, _LANE), 1)
    sub = lax.broadcasted_iota(jnp.int32, (_SUB, _LANE), 0)
    o = jnp.where(lane == 0, c, a)
    o = jnp.where((lane == 0) & (sub == 0), 1, o)
    o_ref[0] = o


def kernel(q, k, v, token_mask):
    B, H, S, D = q.shape
    R = B * H
    sub = S // _LANE
    assert sub == _SUB

    kf = k.reshape(R, sub, _LANE, D)
    q0 = q[:, :, 0, :].reshape(R, 1, D)
    g = _gumbel(B, H, S)
    tm = token_mask.reshape(B, sub, _LANE)

    grid = (R,)
    out = pl.pallas_call(
        _row_body,
        grid=grid,
        in_specs=[
            pl.BlockSpec((1, sub, _LANE, D), lambda r: (r, 0, 0, 0)),
            pl.BlockSpec((1, 1, D), lambda r: (r, 0, 0)),
            pl.BlockSpec((1, sub, _LANE), lambda r: (r, 0, 0)),
            pl.BlockSpec((1, sub, _LANE), lambda r: (r // H, 0, 0)),
        ],
        out_specs=pl.BlockSpec((1, sub, _LANE), lambda r: (r, 0, 0)),
        out_shape=jax.ShapeDtypeStruct((R, sub, _LANE), jnp.int32),
    )(kf, q0, g, tm)
    return out.reshape(B, H, S).astype(jnp.bool_)


# speedup vs baseline: 14.3354x; 14.3354x over previous
import functools

import jax
import jax.numpy as jnp
from jax import lax
from jax.experimental import pallas as pl
from jax.experimental.pallas import tpu as pltpu

TEMPERATURE = 8.0
NUM_SAMPLED = 1024
EPS = 1e-06


@functools.cache
def _gumbel(B, H, S):
    g = jax.random.gumbel(jax.random.key(42), (B, H, S), dtype=jnp.float32)
    return g.reshape(B * H, 1, S)


def _score_body(k_ref, q_ref, g_ref, tm_ref, o_ref):
    S = k_ref.shape[1]
    kb = k_ref[0].astype(jnp.bfloat16)
    qv = q_ref[0].astype(jnp.bfloat16)
    attn = lax.dot_general(
        qv, kb, (((1,), (1,)), ((), ())),
        preferred_element_type=jnp.float32,
    ) / TEMPERATURE
    m = jnp.max(attn)
    e = jnp.exp(attn - m) * tm_ref[0]
    se = jnp.sum(e)
    p = (e + EPS / S) / (se + EPS)
    sc = jnp.log(p) + g_ref[0]

    ki = lax.bitcast_convert_type(sc, jnp.int32)
    t = ki ^ ((ki >> 31) & jnp.int32(0x7FFFFFFF))
    o_ref[0] = lax.bitcast_convert_type(t, jnp.uint32) ^ jnp.uint32(0x80000000)


def _select_body(u_ref, o_ref):
    R = u_ref.shape[0]
    S = u_ref.shape[2]
    u = u_ref[:, 0, :]

    T = jnp.zeros((R, 1), dtype=jnp.uint32)
    for b in range(31, -1, -1):
        cand = T | jnp.uint32(1 << b)
        cnt = jnp.sum((u >= cand).astype(jnp.int32), axis=1, keepdims=True)
        T = jnp.where(cnt >= NUM_SAMPLED, cand, T)

    mask = (u >= T).astype(jnp.int32)
    rolled = pltpu.roll(mask, 1, 1)
    lane = lax.broadcasted_iota(jnp.int32, (R, S), 1)
    o_ref[:, 0, :] = jnp.where(lane == 0, 1, rolled)


def kernel(q, k, v, token_mask):
    B, H, S, D = q.shape
    R = B * H

    kf = k.reshape(R, S, D)
    q0 = q[:, :, 0, :].reshape(R, 1, D)
    g = _gumbel(B, H, S)
    tm = token_mask.reshape(B, 1, S)

    keys = pl.pallas_call(
        _score_body,
        grid=(R,),
        in_specs=[
            pl.BlockSpec((1, S, D), lambda r: (r, 0, 0)),
            pl.BlockSpec((1, 1, D), lambda r: (r, 0, 0)),
            pl.BlockSpec((1, 1, S), lambda r: (r, 0, 0)),
            pl.BlockSpec((1, 1, S), lambda r: (r // H, 0, 0)),
        ],
        out_specs=pl.BlockSpec((1, 1, S), lambda r: (r, 0, 0)),
        out_shape=jax.ShapeDtypeStruct((R, 1, S), jnp.uint32),
    )(kf, q0, g, tm)

    out = pl.pallas_call(
        _select_body,
        in_specs=[pl.BlockSpec((R, 1, S), lambda: (0, 0, 0))],
        out_specs=pl.BlockSpec((R, 1, S), lambda: (0, 0, 0)),
        out_shape=jax.ShapeDtypeStruct((R, 1, S), jnp.int32),
    )(keys)
    return out.reshape(B, H, S).astype(jnp.bool_)
